# baseline (speedup 1.0000x reference)
import functools

import jax
import jax.numpy as jnp
from jax import lax
from jax.experimental import pallas as pl
from jax.experimental.pallas import tpu as pltpu
from jax.experimental.pallas import tpu_sc as plsc

NC, NS, L = 2, 16, 16
NW = NC * NS
B, T, H = 1024, 50, 160
N = B * T
PER_TILE = N // NW
C = 32
NCHUNK = PER_TILE // C
SB = B // NW
HJ = H // L


def _body(ai, e0, e1, e2, e3, ws, bs, wo, bo, wu, bu, wk, bk,
          unk_o, kno_o, obs_o, st_o,
          ai_v, idx0, idx1, idx2, idx3, r0v, r1v, r2v, r3v,
          unk_b, kno_b, obs_b,
          ws_v, bs_v, wo_v, bo_v, wu_v, bu_v, wk_v, bk_v,
          snum_v, sai_v, sidx_v, srows_v, st_b, sem):
    wid = lax.axis_index("s") * NC + lax.axis_index("c")
    lanes = lax.iota(jnp.int32, L)

    pltpu.sync_copy(ws, ws_v)
    pltpu.sync_copy(bs, bs_v)
    pltpu.sync_copy(wo, wo_v)
    pltpu.sync_copy(bo, bo_v)
    pltpu.sync_copy(wu, wu_v)
    pltpu.sync_copy(bu, bu_v)
    pltpu.sync_copy(wk, wk_v)
    pltpu.sync_copy(bk, bk_v)

    b0 = wid * SB
    for g in range(SB // L):
        snum_v[pl.ds(g * L, L)] = (b0 + g * L + lanes) * T
    pltpu.sync_copy(ai.at[snum_v], sai_v)
    for g in range(SB // L):
        v = plsc.load_gather(sai_v, [g * L + lanes, jnp.full((L,), 4, jnp.int32)])
        sidx_v[pl.ds(g * L, L)] = v.astype(jnp.int32)
    pltpu.async_copy(e0.at[sidx_v], srows_v, sem).wait()

    def st_row(r, carry):
        sv = plsc.load_gather(sai_v, [jnp.full((L,), r, jnp.int32),
                                      jnp.full((L,), 3, jnp.int32)])
        s = sv[0]
        rb = r * 2 * H
        for j in range(HJ):
            h16 = j * L
            st_b[pl.ds(rb + h16, L)] = s * ws_v[pl.ds(h16, L)] + bs_v[pl.ds(h16, L)]
            st_b[pl.ds(rb + H + h16, L)] = srows_v[r, pl.ds(h16, L)]
        return carry

    lax.fori_loop(0, SB, st_row, 0)
    pltpu.sync_copy(st_b, st_o.at[pl.ds(b0 * 2 * H, SB * 2 * H)])

    base = wid * PER_TILE

    def chunk(c, carry):
        n0 = base + c * C
        pltpu.sync_copy(ai.at[pl.ds(n0, C)], ai_v)
        for g in range(C // L):
            rr = g * L + lanes
            for k, idxk in enumerate((idx0, idx1, idx2, idx3)):
                v = plsc.load_gather(ai_v, [rr, jnp.full((L,), 4 + k, jnp.int32)])
                idxk[pl.ds(g * L, L)] = v.astype(jnp.int32)
        h0 = pltpu.async_copy(e0.at[idx0], r0v, sem)
        h1 = pltpu.async_copy(e1.at[idx1], r1v, sem)
        h2 = pltpu.async_copy(e2.at[idx2], r2v, sem)
        h3 = pltpu.async_copy(e3.at[idx3], r3v, sem)
        h0.wait()
        h1.wait()
        h2.wait()
        h3.wait()

        def row(r, rcarry):
            sv = plsc.load_gather(ai_v, [jnp.full((L,), r, jnp.int32),
                                         lanes & 7])
            s_o = sv[0]
            s_k1 = sv[1]
            s_k2 = sv[2]
            s_u = sv[3]
            ub = r * 2 * H
            kb = r * 5 * H
            ob = r * H
            for j in range(HJ):
                h16 = j * L
                hl = h16 + lanes
                wuj = wu_v[pl.ds(h16, L)]
                buj = bu_v[pl.ds(h16, L)]
                plsc.store_scatter(unk_b, [ub + 2 * hl], s_u * wuj + buj)
                plsc.store_scatter(unk_b, [ub + 2 * hl + 1], r0v[r, pl.ds(h16, L)])
                wkj = wk_v[pl.ds(h16, L)]
                bkj = bk_v[pl.ds(h16, L)]
                plsc.store_scatter(kno_b, [kb + 5 * hl], s_k1 * wkj + bkj)
                plsc.store_scatter(kno_b, [kb + 5 * hl + 1], s_k2 * wkj + bkj)
                plsc.store_scatter(kno_b, [kb + 5 * hl + 2], r1v[r, pl.ds(h16, L)])
                plsc.store_scatter(kno_b, [kb + 5 * hl + 3], r2v[r, pl.ds(h16, L)])
                plsc.store_scatter(kno_b, [kb + 5 * hl + 4], r3v[r, pl.ds(h16, L)])
                obs_b[pl.ds(ob + h16, L)] = s_o * wo_v[pl.ds(h16, L)] + bo_v[pl.ds(h16, L)]
            return rcarry

        lax.fori_loop(0, C, row, 0)
        pltpu.sync_copy(unk_b, unk_o.at[pl.ds(n0 * 2 * H, C * 2 * H)])
        pltpu.sync_copy(kno_b, kno_o.at[pl.ds(n0 * 5 * H, C * 5 * H)])
        pltpu.sync_copy(obs_b, obs_o.at[pl.ds(n0 * H, C * H)])
        return carry

    lax.fori_loop(0, NCHUNK, chunk, 0)


@jax.jit
def _run(ai2, e0, e1, e2, e3, ws, bs, wo, bo, wu, bu, wk, bk):
    mesh = plsc.VectorSubcoreMesh(core_axis_name="c", subcore_axis_name="s")
    f = pl.kernel(
        _body, mesh=mesh,
        compiler_params=pltpu.CompilerParams(
            needs_layout_passes=False, use_tc_tiling_on_sc=False),
        out_type=[
            jax.ShapeDtypeStruct((N * 2 * H,), jnp.float32),
            jax.ShapeDtypeStruct((N * 5 * H,), jnp.float32),
            jax.ShapeDtypeStruct((N * H,), jnp.float32),
            jax.ShapeDtypeStruct((B * 2 * H,), jnp.float32),
        ],
        scratch_types=[
            pltpu.VMEM((C, 8), jnp.float32),
            pltpu.VMEM((C,), jnp.int32),
            pltpu.VMEM((C,), jnp.int32),
            pltpu.VMEM((C,), jnp.int32),
            pltpu.VMEM((C,), jnp.int32),
            pltpu.VMEM((C, H), jnp.float32),
            pltpu.VMEM((C, H), jnp.float32),
            pltpu.VMEM((C, H), jnp.float32),
            pltpu.VMEM((C, H), jnp.float32),
            pltpu.VMEM((C * 2 * H,), jnp.float32),
            pltpu.VMEM((C * 5 * H,), jnp.float32),
            pltpu.VMEM((C * H,), jnp.float32),
            pltpu.VMEM((H,), jnp.float32),
            pltpu.VMEM((H,), jnp.float32),
            pltpu.VMEM((H,), jnp.float32),
            pltpu.VMEM((H,), jnp.float32),
            pltpu.VMEM((H,), jnp.float32),
            pltpu.VMEM((H,), jnp.float32),
            pltpu.VMEM((H,), jnp.float32),
            pltpu.VMEM((H,), jnp.float32),
            pltpu.VMEM((SB,), jnp.int32),
            pltpu.VMEM((SB, 8), jnp.float32),
            pltpu.VMEM((SB,), jnp.int32),
            pltpu.VMEM((SB, H), jnp.float32),
            pltpu.VMEM((SB * 2 * H,), jnp.float32),
            pltpu.SemaphoreType.DMA,
        ],
    )
    return f(ai2, e0, e1, e2, e3, ws, bs, wo, bo, wu, bu, wk, bk)


def kernel(all_inputs, emb_0, emb_1, emb_2, emb_3, W_static, b_static,
           W_obs, b_obs, W_unknown, b_unknown, W_known, b_known):
    ai2 = all_inputs.reshape(N, 8)
    unk, kno, obs, st = _run(
        ai2, emb_0, emb_1, emb_2, emb_3,
        W_static.reshape(H), b_static, W_obs.reshape(H), b_obs,
        W_unknown.reshape(H), b_unknown, W_known.reshape(H), b_known)
    return (unk.reshape(B, T, H, 2), kno.reshape(B, T, H, 5),
            obs.reshape(B, T, H, 1), st.reshape(B, 2, H))


# speedup vs baseline: 8.1357x; 8.1357x over previous
import jax
import jax.numpy as jnp
from jax import lax
from jax.experimental import pallas as pl
from jax.experimental.pallas import tpu as pltpu
from jax.experimental.pallas import tpu_sc as plsc

NC, NS, L = 2, 16, 16
NW = NC * NS
B, T, H = 1024, 50, 160
N = B * T
BB = 64
NBT = B // BB
UNITS = T * NBT
UPW = UNITS // NW
HT, HI = H // 8, 8
PLANE_BYTES = HT * HI * BB * 4


def _body(idx_t, reg_t, e0, e1, e2, e3, ws, bs, wo, bo, wu, bu, wk, bk,
          unk_o, kno_o, obs_o, st_o,
          idx_v, reg_v,
          r0a, r1a, r2a, r3a, r0b, r1b, r2b, r3b,
          rb0, rb1, rb2, rb3,
          ws_m, bs_m, wo_m, bo_m, wu_m, bu_m, wk_m, bk_m,
          gsem0, gsem1, osem0, osem1, osem2, osem3):
    wid = lax.axis_index("s") * NC + lax.axis_index("c")
    lanes = lax.iota(jnp.int32, L)
    gsem = (gsem0, gsem1)
    osem = (osem0, osem1, osem2, osem3)
    rbufs = (rb0, rb1, rb2, rb3)
    rows = ((r0a, r1a, r2a, r3a), (r0b, r1b, r2b, r3b))
    tabs = (e0, e1, e2, e3)
    bvecs = [g * L + lanes for g in range(BB // L)]

    pltpu.sync_copy(ws, ws_m)
    pltpu.sync_copy(bs, bs_m)
    pltpu.sync_copy(wo, wo_m)
    pltpu.sync_copy(bo, bo_m)
    pltpu.sync_copy(wu, wu_m)
    pltpu.sync_copy(bu, bu_m)
    pltpu.sync_copy(wk, wk_m)
    pltpu.sync_copy(bk, bk_m)

    def unit_tb(u):
        gid = wid * UPW + u
        t = jnp.where(u == UPW, 0, gid // NBT)
        bt = jnp.where(u == UPW, wid, lax.rem(gid, NBT))
        return t, bt

    def fetch(u, d):
        t, bt = unit_tb(u)
        off = t * B + bt * BB
        pltpu.sync_copy(idx_t.at[:, pl.ds(off, BB)], idx_v.at[d])
        pltpu.sync_copy(reg_t.at[:, pl.ds(off, BB)], reg_v.at[d])
        for k in range(4):
            pltpu.async_copy(tabs[k].at[idx_v.at[d, k]], rows[d][k], gsem[d])

    def ring_wait(i):
        pltpu.make_async_copy(
            rbufs[i], kno_o.at[0, 0, :, 0, :, pl.ds(0, BB)], osem[i]).wait()

    def lin_plane(d, rcol, w_m, b_m, ring, dst):
        svs = [reg_v[d, rcol, pl.ds(g * L, L)] for g in range(BB // L)]
        rb = rbufs[ring]

        def htstep(htp, carry):
            w16 = w_m[pl.ds(htp * L, L)]
            b16 = b_m[pl.ds(htp * L, L)]
            for htl in range(2):
                ht = htp * 2 + htl
                for hi in range(8):
                    wh = w16[htl * 8 + hi]
                    bh = b16[htl * 8 + hi]
                    for g in range(BB // L):
                        rb[ht, hi, pl.ds(g * L, L)] = svs[g] * wh + bh
            return carry

        lax.fori_loop(0, HT // 2, htstep, 0)
        pltpu.async_copy(rb, dst, osem[ring])

    def emb_plane(d, k, ring, dst):
        src = rows[d][k]
        rb = rbufs[ring]

        def htstep(ht, carry):
            h8 = ht * 8
            for hi in range(8):
                hv = jnp.full((L,), h8 + hi, jnp.int32)
                for g in range(BB // L):
                    v = plsc.load_gather(src, [bvecs[g], hv])
                    rb[ht, hi, pl.ds(g * L, L)] = v
            return carry

        lax.fori_loop(0, HT, htstep, 0)
        pltpu.async_copy(rb, dst, osem[ring])

    fetch(0, 0)

    def pair(p, carry):
        for d in range(2):
            u = 2 * p + d
            t, bt = unit_tb(u)
            btile = bt // 2
            bi0 = lax.rem(bt, 2) * BB

            @pl.when(u + 1 <= UPW)
            def _():
                fetch(u + 1, 1 - d)

            for k in range(4):
                pltpu.make_async_copy(tabs[k].at[idx_v.at[d, k]],
                                      rows[d][k], gsem[d]).wait()

            @pl.when(u < UPW)
            def _():
                @pl.when(u >= 1)
                def _():
                    ring_wait(0)

                lin_plane(d, 3, wu_m, bu_m, 0,
                          unk_o.at[t, :, :, btile, 0, pl.ds(bi0, BB)])

                @pl.when(u >= 1)
                def _():
                    ring_wait(1)

                lin_plane(d, 1, wk_m, bk_m, 1,
                          kno_o.at[t, 0, :, btile, :, pl.ds(bi0, BB)])

                @pl.when(u >= 1)
                def _():
                    ring_wait(2)

                lin_plane(d, 2, wk_m, bk_m, 2,
                          kno_o.at[t, 1, :, btile, :, pl.ds(bi0, BB)])

                @pl.when(u >= 1)
                def _():
                    ring_wait(3)

                lin_plane(d, 0, wo_m, bo_m, 3,
                          obs_o.at[t, :, :, btile, pl.ds(bi0, BB)])

                ring_wait(0)
                emb_plane(d, 0, 0,
                          unk_o.at[t, :, :, btile, 1, pl.ds(bi0, BB)])
                ring_wait(1)
                emb_plane(d, 1, 1,
                          kno_o.at[t, 2, :, btile, :, pl.ds(bi0, BB)])
                ring_wait(2)
                emb_plane(d, 2, 2,
                          kno_o.at[t, 3, :, btile, :, pl.ds(bi0, BB)])
                ring_wait(3)
                emb_plane(d, 3, 3,
                          kno_o.at[t, 4, :, btile, :, pl.ds(bi0, BB)])

            @pl.when(jnp.logical_and(u == UPW, wid < NBT))
            def _():
                sbtile = wid // 2
                sbi0 = lax.rem(wid, 2) * BB
                ring_wait(0)
                lin_plane(d, 3, ws_m, bs_m, 0,
                          st_o.at[0, :, sbtile, :, pl.ds(sbi0, BB)])
                ring_wait(1)
                emb_plane(d, 0, 1,
                          st_o.at[1, :, sbtile, :, pl.ds(sbi0, BB)])
        return carry

    lax.fori_loop(0, (UPW + 2) // 2, pair, 0)
    ring_wait(0)
    ring_wait(1)
    ring_wait(2)
    ring_wait(3)


@jax.jit
def _run(idx_t, reg_t, e0, e1, e2, e3, ws, bs, wo, bo, wu, bu, wk, bk):
    mesh = plsc.VectorSubcoreMesh(core_axis_name="c", subcore_axis_name="s")
    f = pl.kernel(
        _body, mesh=mesh,
        compiler_params=pltpu.CompilerParams(
            needs_layout_passes=False, use_tc_tiling_on_sc=False),
        out_type=[
            jax.ShapeDtypeStruct((T, HT, HI, NBT // 2, 2, 128), jnp.float32),
            jax.ShapeDtypeStruct((T, 5, HT, NBT // 2, HI, 128), jnp.float32),
            jax.ShapeDtypeStruct((T, HT, HI, NBT // 2, 128), jnp.float32),
            jax.ShapeDtypeStruct((2, HT, NBT // 2, HI, 128), jnp.float32),
        ],
        scratch_types=(
            [pltpu.VMEM((2, 4, BB), jnp.int32),
             pltpu.VMEM((2, 4, BB), jnp.float32)]
            + [pltpu.VMEM((BB, H), jnp.float32)] * 8
            + [pltpu.VMEM((HT, HI, BB), jnp.float32)] * 4
            + [pltpu.VMEM((H,), jnp.float32)] * 8
            + [pltpu.SemaphoreType.DMA] * 6
        ),
    )
    return f(idx_t, reg_t, e0, e1, e2, e3, ws, bs, wo, bo, wu, bu, wk, bk)


def kernel(all_inputs, emb_0, emb_1, emb_2, emb_3, W_static, b_static,
           W_obs, b_obs, W_unknown, b_unknown, W_known, b_known):
    ai_t = all_inputs.transpose(2, 1, 0)
    idx_t = ai_t[4:].astype(jnp.int32).reshape(4, T * B)
    reg_t = ai_t[:4].reshape(4, T * B)
    unk_p, kno_p, obs_p, st_p = _run(
        idx_t, reg_t, emb_0, emb_1, emb_2, emb_3,
        W_static.reshape(H), b_static, W_obs.reshape(H), b_obs,
        W_unknown.reshape(H), b_unknown, W_known.reshape(H), b_known)
    unk = unk_p.transpose(3, 5, 0, 1, 2, 4).reshape(B, T, H, 2)
    kno = kno_p.transpose(3, 5, 0, 2, 4, 1).reshape(B, T, H, 5)
    obs = obs_p.transpose(3, 4, 0, 1, 2).reshape(B, T, H, 1)
    st = st_p.transpose(2, 4, 0, 1, 3).reshape(B, 2, H)
    return (unk, kno, obs, st)


# speedup vs baseline: 10.8032x; 1.3279x over previous
import jax
import jax.numpy as jnp
from jax import lax
from jax.experimental import pallas as pl
from jax.experimental.pallas import tpu as pltpu
from jax.experimental.pallas import tpu_sc as plsc

NC, NS, L = 2, 16, 16
NW = NC * NS
B, T, H = 1024, 50, 160
N = B * T
BB = 64
NBT = B // BB
UNITS = T * NBT
UPW = UNITS // NW
HT, HI = H // 8, 8
PLANE_BYTES = HT * HI * BB * 4


def _body(idx_t, reg_t, e0, e1, e2, e3, ws, bs, wo, bo, wu, bu, wk, bk,
          unk_o, kno_o, obs_o, st_o,
          idx_v, reg_v,
          r0a, r1a, r2a, r3a, r0b, r1b, r2b, r3b,
          rb0, rb1, rb2, rb3,
          ws_m, bs_m, wo_m, bo_m, wu_m, bu_m, wk_m, bk_m,
          gsem0, gsem1, osem0, osem1, osem2, osem3, isem0, isem1):
    wid = lax.axis_index("s") * NC + lax.axis_index("c")
    lanes = lax.iota(jnp.int32, L)
    gsem = (gsem0, gsem1)
    isem = (isem0, isem1)
    osem = (osem0, osem1, osem2, osem3)
    rbufs = (rb0, rb1, rb2, rb3)
    rows = ((r0a, r1a, r2a, r3a), (r0b, r1b, r2b, r3b))
    tabs = (e0, e1, e2, e3)
    bvecs = [g * L + lanes for g in range(BB // L)]

    pltpu.sync_copy(ws, ws_m)
    pltpu.sync_copy(bs, bs_m)
    pltpu.sync_copy(wo, wo_m)
    pltpu.sync_copy(bo, bo_m)
    pltpu.sync_copy(wu, wu_m)
    pltpu.sync_copy(bu, bu_m)
    pltpu.sync_copy(wk, wk_m)
    pltpu.sync_copy(bk, bk_m)

    def unit_tb(u):
        gid = wid * UPW + u
        t = jnp.where(u == UPW, 0, gid // NBT)
        bt = jnp.where(u == UPW, wid, lax.rem(gid, NBT))
        return t, bt

    def fetch_io(u, d):
        t, bt = unit_tb(u)
        off = t * B + bt * BB
        pltpu.async_copy(idx_t.at[:, pl.ds(off, BB)], idx_v.at[d], isem[d])
        pltpu.async_copy(reg_t.at[:, pl.ds(off, BB)], reg_v.at[d], isem[d])

    def fire(d):
        pltpu.make_async_copy(idx_t.at[:, pl.ds(0, BB)], idx_v.at[d],
                              isem[d]).wait()
        pltpu.make_async_copy(reg_t.at[:, pl.ds(0, BB)], reg_v.at[d],
                              isem[d]).wait()
        for k in range(4):
            pltpu.async_copy(tabs[k].at[idx_v.at[d, k]], rows[d][k], gsem[d])

    def ring_wait(i):
        pltpu.make_async_copy(
            rbufs[i], kno_o.at[0, 0, :, 0, :, pl.ds(0, BB)], osem[i]).wait()

    def lin_plane(d, rcol, w_m, b_m, ring, dst):
        svs = [reg_v[d, rcol, pl.ds(g * L, L)] for g in range(BB // L)]
        rb = rbufs[ring]

        @plsc.parallel_loop(0, HT // 2, unroll=1)
        def _(htp):
            w16 = w_m[pl.ds(htp * L, L)]
            b16 = b_m[pl.ds(htp * L, L)]
            for htl in range(2):
                ht = htp * 2 + htl
                for hi in range(8):
                    wh = w16[htl * 8 + hi]
                    bh = b16[htl * 8 + hi]
                    for g in range(BB // L):
                        rb[ht, hi, pl.ds(g * L, L)] = svs[g] * wh + bh

        pltpu.async_copy(rb, dst, osem[ring])

    def emb_plane(d, k, ring, dst):
        src = rows[d][k]
        rb = rbufs[ring]

        @plsc.parallel_loop(0, HT, unroll=1)
        def _(ht):
            h8 = ht * 8
            for hi in range(8):
                hv = jnp.full((L,), h8 + hi, jnp.int32)
                for g in range(BB // L):
                    v = plsc.load_gather(src, [bvecs[g], hv])
                    rb[ht, hi, pl.ds(g * L, L)] = v

        pltpu.async_copy(rb, dst, osem[ring])

    fetch_io(0, 0)
    fire(0)
    fetch_io(1, 1)

    def pair(p, carry):
        for d in range(2):
            u = 2 * p + d
            t, bt = unit_tb(u)
            btile = bt // 2
            bi0 = lax.rem(bt, 2) * BB

            @pl.when(u + 1 <= UPW)
            def _():
                fire(1 - d)

            for k in range(4):
                pltpu.make_async_copy(tabs[k].at[idx_v.at[d, k]],
                                      rows[d][k], gsem[d]).wait()

            @pl.when(u < UPW)
            def _():
                @pl.when(u >= 1)
                def _():
                    ring_wait(0)

                lin_plane(d, 3, wu_m, bu_m, 0,
                          unk_o.at[t, :, :, btile, 0, pl.ds(bi0, BB)])

                @pl.when(u >= 1)
                def _():
                    ring_wait(1)

                lin_plane(d, 1, wk_m, bk_m, 1,
                          kno_o.at[t, 0, :, btile, :, pl.ds(bi0, BB)])

                @pl.when(u >= 1)
                def _():
                    ring_wait(2)

                lin_plane(d, 2, wk_m, bk_m, 2,
                          kno_o.at[t, 1, :, btile, :, pl.ds(bi0, BB)])

                @pl.when(u >= 1)
                def _():
                    ring_wait(3)

                lin_plane(d, 0, wo_m, bo_m, 3,
                          obs_o.at[t, :, :, btile, pl.ds(bi0, BB)])

                ring_wait(0)
                emb_plane(d, 0, 0,
                          unk_o.at[t, :, :, btile, 1, pl.ds(bi0, BB)])
                ring_wait(1)
                emb_plane(d, 1, 1,
                          kno_o.at[t, 2, :, btile, :, pl.ds(bi0, BB)])
                ring_wait(2)
                emb_plane(d, 2, 2,
                          kno_o.at[t, 3, :, btile, :, pl.ds(bi0, BB)])
                ring_wait(3)
                emb_plane(d, 3, 3,
                          kno_o.at[t, 4, :, btile, :, pl.ds(bi0, BB)])

            @pl.when(u + 2 <= UPW)
            def _():
                fetch_io(u + 2, d)

            @pl.when(jnp.logical_and(u == UPW, wid < NBT))
            def _():
                sbtile = wid // 2
                sbi0 = lax.rem(wid, 2) * BB
                ring_wait(0)
                lin_plane(d, 3, ws_m, bs_m, 0,
                          st_o.at[0, :, sbtile, :, pl.ds(sbi0, BB)])
                ring_wait(1)
                emb_plane(d, 0, 1,
                          st_o.at[1, :, sbtile, :, pl.ds(sbi0, BB)])
        return carry

    lax.fori_loop(0, (UPW + 2) // 2, pair, 0)
    ring_wait(0)
    ring_wait(1)
    ring_wait(2)
    ring_wait(3)


@jax.jit
def _run(idx_t, reg_t, e0, e1, e2, e3, ws, bs, wo, bo, wu, bu, wk, bk):
    mesh = plsc.VectorSubcoreMesh(core_axis_name="c", subcore_axis_name="s")
    f = pl.kernel(
        _body, mesh=mesh,
        compiler_params=pltpu.CompilerParams(
            needs_layout_passes=False, use_tc_tiling_on_sc=False),
        out_type=[
            jax.ShapeDtypeStruct((T, HT, HI, NBT // 2, 2, 128), jnp.float32),
            jax.ShapeDtypeStruct((T, 5, HT, NBT // 2, HI, 128), jnp.float32),
            jax.ShapeDtypeStruct((T, HT, HI, NBT // 2, 128), jnp.float32),
            jax.ShapeDtypeStruct((2, HT, NBT // 2, HI, 128), jnp.float32),
        ],
        scratch_types=(
            [pltpu.VMEM((2, 4, BB), jnp.int32),
             pltpu.VMEM((2, 4, BB), jnp.float32)]
            + [pltpu.VMEM((BB, H), jnp.float32)] * 8
            + [pltpu.VMEM((HT, HI, BB), jnp.float32)] * 4
            + [pltpu.VMEM((H,), jnp.float32)] * 8
            + [pltpu.SemaphoreType.DMA] * 8
        ),
    )
    return f(idx_t, reg_t, e0, e1, e2, e3, ws, bs, wo, bo, wu, bu, wk, bk)


def kernel(all_inputs, emb_0, emb_1, emb_2, emb_3, W_static, b_static,
           W_obs, b_obs, W_unknown, b_unknown, W_known, b_known):
    ai_t = all_inputs.transpose(2, 1, 0)
    idx_t = ai_t[4:].astype(jnp.int32).reshape(4, T * B)
    reg_t = ai_t[:4].reshape(4, T * B)
    unk_p, kno_p, obs_p, st_p = _run(
        idx_t, reg_t, emb_0, emb_1, emb_2, emb_3,
        W_static.reshape(H), b_static, W_obs.reshape(H), b_obs,
        W_unknown.reshape(H), b_unknown, W_known.reshape(H), b_known)
    unk = unk_p.transpose(3, 5, 0, 1, 2, 4).reshape(B, T, H, 2)
    kno = kno_p.transpose(3, 5, 0, 2, 4, 1).reshape(B, T, H, 5)
    obs = obs_p.transpose(3, 4, 0, 1, 2).reshape(B, T, H, 1)
    st = st_p.transpose(2, 4, 0, 1, 3).reshape(B, 2, H)
    return (unk, kno, obs, st)
